# Initial kernel scaffold; baseline (speedup 1.0000x reference)
import jax
import jax.numpy as jnp
from jax.experimental import pallas as pl


def kernel(h, x, v, params):
    raise NotImplementedError("write your pallas kernel here")


# speedup vs baseline: 1.1185x; 1.1185x over previous
import functools

import jax
import jax.numpy as jnp
from jax.experimental import pallas as pl

N = 256
F = 64
H = 4
NRBF = 50
BLK = 32


def _silu(x):
    return x * jax.nn.sigmoid(x)


def _sake_block_kernel(
    h_ref, x_ref, xt_ref, v_ref,
    win_j_ref, win_i_ref, bin_ref, means_ref, betas_ref,
    w0j_ref, w0i_ref, w0x_ref, w0d_ref, b0_ref,
    w1_ref, b1_ref, wsem_ref, bsem_ref, gamma_ref,
    wx3_ref, wp0_ref, bp0_ref, wp1_ref, bp1_ref, vmix_ref,
    wn0h_ref, wn0he_ref, wn0hc_ref, bn0_ref, wn1_ref, bn1_ref,
    wv0_ref, bv0_ref, wv1_ref,
    hn_ref, xn_ref, vn_ref,
):
    B = BLK
    i0 = pl.program_id(0) * B
    f32 = jnp.float32

    h_all = h_ref[...]
    h_blk = h_ref[pl.ds(i0, B), :]
    x_blk = x_ref[pl.ds(i0, B), :]
    v_blk = v_ref[...]

    dx = [xt_ref[d:d + 1, :] - x_ref[pl.ds(i0, B), d:d + 1] for d in range(3)]
    d2 = dx[0] * dx[0] + dx[1] * dx[1] + dx[2] * dx[2]
    dist = jnp.sqrt(jnp.maximum(d2, 0.0) + 1e-10)

    a_j = jnp.dot(h_all, win_j_ref[...], preferred_element_type=f32)
    b_i = jnp.dot(h_blk, win_i_ref[...], preferred_element_type=f32)
    e1 = a_j[None, :, :] + b_i[:, None, :] + bin_ref[...][None, :, :]

    cut = 0.5 * (jnp.cos(dist * (jnp.pi / 5.0)) + 1.0)
    cut = cut * (dist < 5.0).astype(f32)
    ed = jnp.exp(-dist)
    t = ed[:, :, None] - means_ref[...][None, :, :]
    rbf = cut[:, :, None] * jnp.exp(-betas_ref[...][None, :, :] * t * t)
    xfeat = rbf * e1

    E = B * N
    c_j = jnp.dot(h_all, w0j_ref[...], preferred_element_type=f32)
    d_i = jnp.dot(h_blk, w0i_ref[...], preferred_element_type=f32)
    hc = (c_j[None, :, :] + d_i[:, None, :]).reshape(E, F)
    pre0 = (jnp.dot(xfeat.reshape(E, NRBF), w0x_ref[...],
                    preferred_element_type=f32)
            + hc
            + (dist[:, :, None] * w0d_ref[...][None, :, :]).reshape(E, F)
            + b0_ref[...])
    y = _silu(pre0)
    hem = jnp.dot(y, w1_ref[...], preferred_element_type=f32) + b1_ref[...]
    hem3 = hem.reshape(B, N, F)

    semp = (jnp.dot(hem, wsem_ref[...], preferred_element_type=f32)
            + bsem_ref[...]).reshape(B, N, H)
    cols = jax.lax.broadcasted_iota(jnp.int32, (B, N), 1)
    rows = jax.lax.broadcasted_iota(jnp.int32, (B, N), 0) + i0
    eye = (cols == rows).astype(f32)
    neg_masked_dist = -(dist + 1e5 * eye)

    acc = jnp.zeros((E, H * F), dtype=f32)
    he_contrib = jnp.zeros((B, F), dtype=f32)
    for hd in range(H):
        s = semp[:, :, hd]
        s = jnp.where(s > 0, s, 2.0 * (jnp.exp(s * 0.5) - 1.0))
        s = s - 1e5 * eye
        s = s - jnp.max(s, axis=1, keepdims=True)
        s = jnp.exp(s)
        s = s / jnp.sum(s, axis=1, keepdims=True)

        eu = neg_masked_dist * gamma_ref[0, hd]
        eu = eu - jnp.max(eu, axis=1, keepdims=True)
        eu = jnp.exp(eu)
        eu = eu / jnp.sum(eu, axis=1, keepdims=True)

        comb = eu * s
        comb = comb / jnp.sum(comb, axis=1, keepdims=True)

        scaled = hem3 * comb[:, :, None]
        acc = acc + jnp.dot(scaled.reshape(E, F), wx3_ref[:, hd, :],
                            preferred_element_type=f32)
        he_contrib = he_contrib + jnp.dot(
            jnp.sum(scaled, axis=1), wn0he_ref[:, hd, :],
            preferred_element_type=f32)

    coeff = jnp.tanh(acc).reshape(B, N, H * F)

    inv = 1.0 / (dist + 1e-5)
    cs = [jnp.sum(coeff * (dx[d] * inv)[:, :, None], axis=1) * (1.0 / N)
          for d in range(3)]

    cnorm = cs[0] * cs[0] + cs[1] * cs[1] + cs[2] * cs[2]
    hcomb = _silu(jnp.dot(cnorm, wp0_ref[...], preferred_element_type=f32)
                  + bp0_ref[...])
    hcomb = _silu(jnp.dot(hcomb, wp1_ref[...], preferred_element_type=f32)
                  + bp1_ref[...])

    dv = jnp.concatenate(
        [jnp.dot(cs[d], vmix_ref[...], preferred_element_type=f32)
         for d in range(3)], axis=1)

    pre = (jnp.dot(h_blk, wn0h_ref[...], preferred_element_type=f32)
           + he_contrib
           + jnp.dot(hcomb, wn0hc_ref[...], preferred_element_type=f32)
           + bn0_ref[...])
    o = _silu(pre)
    o = _silu(jnp.dot(o, wn1_ref[...], preferred_element_type=f32)
              + bn1_ref[...])
    h_new = h_blk + o

    sc = _silu(jnp.dot(h_new, wv0_ref[...], preferred_element_type=f32)
               + bv0_ref[...])
    sc = 2.0 * jax.nn.sigmoid(jnp.dot(sc, wv1_ref[...],
                                      preferred_element_type=f32))
    v_new = dv + sc * v_blk
    x_new = x_blk + v_new

    hn_ref[...] = h_new
    xn_ref[...] = x_new
    vn_ref[...] = v_new


@jax.jit
def kernel(h, x, v, params):
    ep = params["edge_model"]
    win = ep["mlp_in"]["w"]
    w0 = ep["mlp_out0"]["w"]
    wn0 = params["node_mlp0"]["w"]

    def row(b):
        return b.reshape(1, -1)

    ins = [
        h, x, x.T, v,
        win[:F], win[F:], row(ep["mlp_in"]["b"]),
        row(ep["kernel"]["means"]), row(ep["kernel"]["betas"]),
        w0[:F], w0[F:2 * F], w0[2 * F:2 * F + NRBF], w0[2 * F + NRBF:],
        row(ep["mlp_out0"]["b"]),
        ep["mlp_out1"]["w"], row(ep["mlp_out1"]["b"]),
        params["semantic_attention_mlp"]["w"],
        row(params["semantic_attention_mlp"]["b"]),
        row(jnp.exp(params["log_gamma"])),
        params["x_mixing"]["w"].reshape(F, H, H * F),
        params["post_norm_mlp0"]["w"], row(params["post_norm_mlp0"]["b"]),
        params["post_norm_mlp1"]["w"], row(params["post_norm_mlp1"]["b"]),
        params["v_mixing"]["w"],
        wn0[:F], wn0[F:F + H * F].reshape(F, H, F), wn0[F + H * F:],
        row(params["node_mlp0"]["b"]),
        params["node_mlp1"]["w"], row(params["node_mlp1"]["b"]),
        params["velocity_mlp0"]["w"], row(params["velocity_mlp0"]["b"]),
        params["velocity_mlp1"]["w"],
    ]

    def full(a):
        return pl.BlockSpec(a.shape, lambda i: (0,) * a.ndim)

    in_specs = [full(a) for a in ins]
    in_specs[3] = pl.BlockSpec((BLK, 3), lambda i: (i, 0))

    out_shape = [
        jax.ShapeDtypeStruct((N, F), jnp.float32),
        jax.ShapeDtypeStruct((N, 3), jnp.float32),
        jax.ShapeDtypeStruct((N, 3), jnp.float32),
    ]
    out_specs = [
        pl.BlockSpec((BLK, F), lambda i: (i, 0)),
        pl.BlockSpec((BLK, 3), lambda i: (i, 0)),
        pl.BlockSpec((BLK, 3), lambda i: (i, 0)),
    ]

    h_new, x_new, v_new = pl.pallas_call(
        _sake_block_kernel,
        grid=(N // BLK,),
        in_specs=in_specs,
        out_specs=out_specs,
        out_shape=out_shape,
    )(*ins)
    return h_new, x_new, v_new
